# Initial kernel scaffold; baseline (speedup 1.0000x reference)
import jax
import jax.numpy as jnp
from jax.experimental import pallas as pl


def kernel(features, indices, weight, bias):
    raise NotImplementedError("write your pallas kernel here")


# speedup vs baseline: 4.0071x; 4.0071x over previous
import jax
import jax.numpy as jnp
from jax.experimental import pallas as pl

SPATIAL = (128, 128, 128)
N_BLOCK = 2000


def _pack(indices):
    D0, D1, D2 = SPATIAL
    vol = D0 * D1 * D2
    return (indices[:, 0] * vol + indices[:, 1] * (D1 * D2)
            + indices[:, 2] * D2 + indices[:, 3]).astype(jnp.int32)


def _matmul_body(g_ref, w_ref, b_ref, o_ref):
    o_ref[...] = (jnp.dot(g_ref[...], w_ref[...],
                          preferred_element_type=jnp.float32)
                  + b_ref[...])


def kernel(features, indices, weight, bias):
    N, C_IN = features.shape
    C_OUT = weight.shape[0]
    packed = _pack(indices)
    order = jnp.argsort(packed)
    sorted_pack = packed[order]
    fs = features[order]

    gathered = []
    wcols = []
    for k0 in range(3):
        for k1 in range(3):
            for k2 in range(3):
                delta = (k0 - 1) * (128 * 128) + (k1 - 1) * 128 + (k2 - 1)
                q = packed + jnp.int32(delta)
                p = jnp.searchsorted(sorted_pack, q)
                pc = jnp.minimum(p, N - 1)
                ok = (p < N) & (sorted_pack[pc] == q)
                rows = jnp.where(ok[:, None], fs[pc], 0.0)
                gathered.append(rows)
                wcols.append(weight[:, k0, k1, k2, :].T)
    G = jnp.concatenate(gathered, axis=1)
    W = jnp.concatenate(wcols, axis=0)

    K = G.shape[1]
    grid = (N // N_BLOCK,)
    out = pl.pallas_call(
        _matmul_body,
        grid=grid,
        in_specs=[
            pl.BlockSpec((N_BLOCK, K), lambda i: (i, 0)),
            pl.BlockSpec((K, C_OUT), lambda i: (0, 0)),
            pl.BlockSpec((1, C_OUT), lambda i: (0, 0)),
        ],
        out_specs=pl.BlockSpec((N_BLOCK, C_OUT), lambda i: (i, 0)),
        out_shape=jax.ShapeDtypeStruct((N, C_OUT), jnp.float32),
    )(G, W, bias[None, :])
    return out
